# baseline (speedup 1.0000x reference)
import jax
import jax.numpy as jnp
from jax import lax
from jax.experimental import pallas as pl
from jax.experimental.pallas import tpu as pltpu
from jax.experimental.pallas import tpu_sc as plsc

B = 16384
F = 26
V = 100000
NC = 2
NS = 16
NW = NC * NS
BPW = B // NW
IPW = F * BPW
GN = 16 * BPW
U = 4


def _sc_body(x_hbm, w_hbm, bias_hbm, out_hbm,
             idx_v, vals_v, g_v, g2d_v, out_v, bias_v, spm, sem):
    sid = lax.axis_index("s")
    wid = sid * NC + lax.axis_index("c")
    pltpu.sync_copy(x_hbm.at[wid], idx_v)
    pltpu.sync_copy(bias_hbm, bias_v)

    pltpu.async_copy(w_hbm.at[idx_v], vals_v.at[pl.ds(0, IPW)], sem).wait()

    zeros = jnp.zeros((16,), jnp.float32)
    vals_v[pl.ds(IPW, 16)] = zeros
    g_v[pl.ds(GN, 16)] = zeros
    mask10 = jnp.where(lax.iota(jnp.int32, 16) < (F - 16), 1.0, 0.0)
    bias_vec = bias_v[...]

    def fold(gi, carry):
        for u in range(U):
            j = gi * U + u
            va = vals_v[pl.ds(j * F, 16)]
            vb = vals_v[pl.ds(j * F + 16, 16)]
            g_v[pl.ds(j * 16, 16)] = va + vb * mask10
        return carry
    lax.fori_loop(0, BPW // U, fold, 0)

    for d in (8, 4, 2):
        def level(gi, carry, d=d):
            for u in range(U):
                base = (gi * U + u) * 16
                va = g_v[pl.ds(base, 16)]
                vb = g_v[pl.ds(base + d, 16)]
                g_v[pl.ds(base, 16)] = va + vb
            return carry
        lax.fori_loop(0, BPW // U, level, 0)

    def last(gi, carry):
        for u in range(U):
            i = gi * U + u
            va = g_v[pl.ds(i * 16, 16)]
            vb = g_v[pl.ds(i * 16 + 1, 16)]
            g2d_v[i, :] = va + vb + bias_vec
        return carry
    lax.fori_loop(0, BPW // U, last, 0)

    pltpu.sync_copy(g2d_v.at[:, 0], spm.at[pl.ds(sid * BPW, BPW)])
    pltpu.sync_copy(spm.at[pl.ds(sid * BPW, BPW)], out_v)
    pltpu.sync_copy(out_v, out_hbm.at[wid])


def kernel(x_sparse, W, bias):
    xf = x_sparse.astype(jnp.int32) + jnp.arange(F, dtype=jnp.int32) * V
    x2 = xf.reshape(NW, IPW)
    wflat = W.reshape(-1)
    bias16 = jnp.broadcast_to(bias.astype(jnp.float32), (16,))
    mesh = plsc.VectorSubcoreMesh(core_axis_name="c", subcore_axis_name="s")
    out = pl.kernel(
        _sc_body,
        out_type=jax.ShapeDtypeStruct((NW, BPW), jnp.float32),
        mesh=mesh,
        scratch_types=[
            pltpu.VMEM((IPW,), jnp.int32),
            pltpu.VMEM((IPW + 16,), jnp.float32),
            pltpu.VMEM((GN + 16,), jnp.float32),
            pltpu.VMEM((BPW, 16), jnp.float32),
            pltpu.VMEM((BPW,), jnp.float32),
            pltpu.VMEM((16,), jnp.float32),
            pltpu.VMEM_SHARED((NS * BPW,), jnp.float32),
            pltpu.SemaphoreType.DMA,
        ],
    )(x2, wflat, bias16)
    return out.reshape(B, 1)
